# baseline (speedup 1.0000x reference)
import functools

import jax
import jax.numpy as jnp
from jax.experimental import pallas as pl

F32 = jnp.float32
NUM_Q = 4
STRIDE_T = 2
DGR = 3
DEPTH = 3


def _conv1d_x(x, p, stride=1, padding=0, dilation=1):
    out = jax.lax.conv_general_dilated(x, p['w'], (stride,), [(padding, padding)],
                                       rhs_dilation=(dilation,),
                                       dimension_numbers=('NCH', 'OIH', 'NCH'))
    return out + p['b'][None, :, None]


def _resblock_x(x, p, dilation):
    h = jax.nn.relu(x)
    h = _conv1d_x(h, p['c1'], 1, dilation, dilation)
    h = jax.nn.relu(h)
    h = _conv1d_x(h, p['c2'], 1, 0, 1)
    return x + h


def _encoder_x(x, ep):
    x = jax.nn.relu(_conv1d_x(x, ep['conv_in'], 1, 1))
    for blk in ep['downs']:
        x = _conv1d_x(x, blk['down'], STRIDE_T, STRIDE_T // 2)
        for p, dil in zip(blk['res'], [DGR ** d for d in range(DEPTH)]):
            x = _resblock_x(x, p, dil)
    x = _conv1d_x(x, ep['conv_out'], 1, 1)
    return x


def _dot(a, b):
    return jax.lax.dot_general(a, b, (((1,), (0,)), ((), ())),
                               preferred_element_type=F32)


def _conv3(v, wref, bref, dil, T):
    xp = jnp.pad(v, ((dil, dil), (0, 0)))
    acc = jnp.broadcast_to(bref[...], (T, wref.shape[-1])).astype(F32)
    for k in range(3):
        acc = acc + _dot(xp[k * dil:k * dil + T, :], wref[k])
    return acc


def _conv1(v, wref, bref):
    return _dot(v, wref[0]) + bref[...]


def _up(v, wref, bref, T):
    C = wref.shape[-1]
    xp = jnp.pad(v, ((1, 1), (0, 0)))
    b = jnp.broadcast_to(bref[...], (T, C)).astype(F32)
    ye = b + _dot(xp[0:T], wref[0]) + _dot(xp[1:T + 1], wref[1])
    yo = b + _dot(xp[1:T + 1], wref[2]) + _dot(xp[2:T + 2], wref[3])
    return jnp.concatenate([ye[:, None, :], yo[:, None, :]], axis=1).reshape(2 * T, C)


def _resblock(v, w1, b1, w2, b2, dil, T):
    h = jnp.maximum(v, 0.0)
    h = _conv3(h, w1, b1, dil, T)
    h = jnp.maximum(h, 0.0)
    h = _conv1(h, w2, b2)
    return v + h


def _res_chain(v, refs, dils, T):
    for i, d in enumerate(dils):
        v = _resblock(v, refs[4 * i], refs[4 * i + 1], refs[4 * i + 2],
                      refs[4 * i + 3], d, T)
    return v


def _loop(nb, fn):
    jax.lax.fori_loop(0, nb, lambda b, c: (fn(b), 0)[1], 0)


def _lfq_body(x_ref, *rest):
    lw = rest[:4 * NUM_Q]
    q_ref, loss_ref = rest[4 * NUM_Q], rest[4 * NUM_Q + 1]
    r = x_ref[...]
    acc = jnp.zeros_like(r)
    for l in range(NUM_Q):
        winT, bin_, woutT, bout = lw[4 * l:4 * l + 4]
        z = _dot(r, winT[...]) + bin_[...]
        q = jnp.where(z > 0, 1.0, -1.0).astype(F32)
        out = _dot(q, woutT[...]) + bout[...]
        r = r - out
        acc = acc + out
        loss_ref[0:1, l:l + 1] = jnp.mean((z - q) ** 2).reshape(1, 1)
    q_ref[...] = acc


def _dec1_body(nb, x_ref, wi, bi, *rest):
    res = rest[:12]
    wu, bu = rest[12], rest[13]
    o_ref = rest[14]

    def one(b):
        v = jnp.maximum(_conv3(x_ref[b], wi, bi, 1, 32), 0.0)
        v = _res_chain(v, res, (9, 3, 1), 32)
        v = _up(v, wu, bu, 32)
        o_ref[b] = v
    _loop(nb, one)


def _dec2_body(nb, x_ref, *rest):
    res = rest[:12]
    wu, bu = rest[12], rest[13]
    o_ref = rest[14]

    def one(b):
        v = _res_chain(x_ref[b], res, (9, 3, 1), 64)
        v = _up(v, wu, bu, 64)
        o_ref[b] = v
    _loop(nb, one)


def _dec3_body(nb, x_ref, *rest):
    res = rest[:12]
    wu, bu, w1, b1, w2, b2 = rest[12:18]
    o_ref = rest[18]

    def one(b):
        v = _res_chain(x_ref[b], res, (9, 3, 1), 128)
        v = _up(v, wu, bu, 128)
        v = jnp.maximum(_conv3(v, w1, b1, 1, 256), 0.0)
        v = _conv3(v, w2, b2, 1, 256)
        o_ref[b] = v
    _loop(nb, one)


def _const_spec(w):
    nd = w.ndim
    return pl.BlockSpec(w.shape, (lambda nd: (lambda i: (0,) * nd))(nd))


def _seg(body, x, weights, T_out, C_out, nb):
    B = x.shape[0]
    in_specs = [pl.BlockSpec((nb,) + x.shape[1:], lambda i: (i, 0, 0))]
    in_specs += [_const_spec(w) for w in weights]
    return pl.pallas_call(
        functools.partial(body, nb),
        grid=(B // nb,),
        in_specs=in_specs,
        out_specs=pl.BlockSpec((nb, T_out, C_out), lambda i: (i, 0, 0)),
        out_shape=jax.ShapeDtypeStruct((B, T_out, C_out), F32),
    )(x, *weights)


def _lfq_call(x2d, weights):
    n = x2d.shape[0]
    in_specs = [pl.BlockSpec(x2d.shape, lambda: (0, 0))]
    in_specs += [pl.BlockSpec(w.shape, (lambda nd: (lambda: (0,) * nd))(w.ndim))
                 for w in weights]
    return pl.pallas_call(
        _lfq_body,
        in_specs=in_specs,
        out_specs=[pl.BlockSpec(x2d.shape, lambda: (0, 0)),
                   pl.BlockSpec((1, NUM_Q), lambda: (0, 0))],
        out_shape=[jax.ShapeDtypeStruct((n, 512), F32),
                   jax.ShapeDtypeStruct((1, NUM_Q), F32)],
    )(x2d, *weights)


def _wt(p):
    return p['w'].transpose(2, 1, 0)


def _bt(p):
    return p['b'].reshape(1, -1)


def _res_w(blocks):
    out = []
    for rb in blocks:
        out += [_wt(rb['c1']), _bt(rb['c1']), _wt(rb['c2']), _bt(rb['c2'])]
    return out


def _up_w(p):
    wt = _wt(p)
    return jnp.stack([wt[0], wt[1] + wt[2], wt[0] + wt[1], wt[2]])


def kernel(x, params):
    enc, dec, lfq = params['encoder'], params['decoder'], params['lfq']
    B = x.shape[0]

    x_in = jnp.transpose(x, (0, 2, 1)).astype(F32)
    x_enc = _encoder_x(x_in, enc)
    x_enc = jnp.transpose(x_enc, (0, 2, 1))

    lfq_w = []
    for p in lfq:
        lfq_w += [p['win'].T, p['bin'].reshape(1, -1),
                  p['wout'].T, p['bout'].reshape(1, -1)]
    x_q2d, losses = _lfq_call(x_enc.reshape(B * 32, 512), lfq_w)
    x_q = x_q2d.reshape(B, 32, 512)
    commit_loss = losses.reshape(NUM_Q)
    perplexity = jnp.sum(commit_loss)

    v = _seg(_dec1_body, x_q,
             [_wt(dec['conv_in']), _bt(dec['conv_in'])]
             + _res_w(dec['ups'][0]['res'])
             + [_up_w(dec['ups'][0]['up']), _bt(dec['ups'][0]['up'])], 64, 512, 8)
    v = _seg(_dec2_body, v,
             _res_w(dec['ups'][1]['res'])
             + [_up_w(dec['ups'][1]['up']), _bt(dec['ups'][1]['up'])], 128, 512, 8)
    v = _seg(_dec3_body, v,
             _res_w(dec['ups'][2]['res'])
             + [_up_w(dec['ups'][2]['up']), _bt(dec['ups'][2]['up']),
                _wt(dec['conv1']), _bt(dec['conv1']),
                _wt(dec['conv2']), _bt(dec['conv2'])], 256, 263, 8)

    x_out = jnp.transpose(v, (0, 2, 1))
    return x_out, commit_loss, perplexity


# speedup vs baseline: 1.4538x; 1.4538x over previous
import functools

import jax
import jax.numpy as jnp
from jax.experimental import pallas as pl

F32 = jnp.float32
NUM_Q = 4
STRIDE_T = 2
DGR = 3
DEPTH = 3


def _conv1d_x(x, p, stride=1, padding=0, dilation=1):
    out = jax.lax.conv_general_dilated(x, p['w'], (stride,), [(padding, padding)],
                                       rhs_dilation=(dilation,),
                                       dimension_numbers=('NCH', 'OIH', 'NCH'))
    return out + p['b'][None, :, None]


def _resblock_x(x, p, dilation):
    h = jax.nn.relu(x)
    h = _conv1d_x(h, p['c1'], 1, dilation, dilation)
    h = jax.nn.relu(h)
    h = _conv1d_x(h, p['c2'], 1, 0, 1)
    return x + h


def _encoder_x(x, ep):
    x = jax.nn.relu(_conv1d_x(x, ep['conv_in'], 1, 1))
    for blk in ep['downs']:
        x = _conv1d_x(x, blk['down'], STRIDE_T, STRIDE_T // 2)
        for p, dil in zip(blk['res'], [DGR ** d for d in range(DEPTH)]):
            x = _resblock_x(x, p, dil)
    x = _conv1d_x(x, ep['conv_out'], 1, 1)
    return x


def _dot(a, b):
    return jax.lax.dot_general(a, b, (((1,), (0,)), ((), ())),
                               preferred_element_type=F32)


def _conv3(v, wref, bref, dil, nb, T):
    C = wref.shape[-1]
    xp = jnp.pad(v, ((0, 0), (dil, dil), (0, 0)))
    acc = jnp.broadcast_to(bref[...], (nb * T, C)).astype(F32)
    for k in range(3):
        acc = acc + _dot(xp[:, k * dil:k * dil + T, :].reshape(nb * T, -1), wref[k])
    return acc.reshape(nb, T, C)


def _conv1(v, wref, bref, nb, T):
    C = wref.shape[-1]
    return (_dot(v.reshape(nb * T, -1), wref[0]) + bref[...]).reshape(nb, T, C)


def _up(v, wref, bref, nb, T):
    C = wref.shape[-1]
    xp = jnp.pad(v, ((0, 0), (1, 1), (0, 0)))
    b = jnp.broadcast_to(bref[...], (nb * T, C)).astype(F32)
    x0 = xp[:, 0:T, :].reshape(nb * T, -1)
    x1 = xp[:, 1:T + 1, :].reshape(nb * T, -1)
    x2 = xp[:, 2:T + 2, :].reshape(nb * T, -1)
    ye = (b + _dot(x0, wref[0]) + _dot(x1, wref[1])).reshape(nb, T, 1, C)
    yo = (b + _dot(x1, wref[2]) + _dot(x2, wref[3])).reshape(nb, T, 1, C)
    return jnp.concatenate([ye, yo], axis=2).reshape(nb, 2 * T, C)


def _resblock(v, w1, b1, w2, b2, dil, nb, T):
    h = jnp.maximum(v, 0.0)
    h = _conv3(h, w1, b1, dil, nb, T)
    h = jnp.maximum(h, 0.0)
    h = _conv1(h, w2, b2, nb, T)
    return v + h


def _res_chain(v, refs, dils, nb, T):
    for i, d in enumerate(dils):
        v = _resblock(v, refs[4 * i], refs[4 * i + 1], refs[4 * i + 2],
                      refs[4 * i + 3], d, nb, T)
    return v


def _lfq_body(x_ref, *rest):
    lw = rest[:4 * NUM_Q]
    q_ref, loss_ref = rest[4 * NUM_Q], rest[4 * NUM_Q + 1]
    r = x_ref[...]
    acc = jnp.zeros_like(r)
    for l in range(NUM_Q):
        winT, bin_, woutT, bout = lw[4 * l:4 * l + 4]
        z = _dot(r, winT[...]) + bin_[...]
        q = jnp.where(z > 0, 1.0, -1.0).astype(F32)
        out = _dot(q, woutT[...]) + bout[...]
        r = r - out
        acc = acc + out
        loss_ref[0:1, l:l + 1] = jnp.mean((z - q) ** 2).reshape(1, 1)
    q_ref[...] = acc


def _dec1_body(nb, x_ref, wi, bi, *rest):
    res = rest[:12]
    wu, bu = rest[12], rest[13]
    o_ref = rest[14]
    v = jnp.maximum(_conv3(x_ref[...], wi, bi, 1, nb, 32), 0.0)
    v = _res_chain(v, res, (9, 3, 1), nb, 32)
    o_ref[...] = _up(v, wu, bu, nb, 32)


def _dec2_body(nb, x_ref, *rest):
    res = rest[:12]
    wu, bu = rest[12], rest[13]
    o_ref = rest[14]
    v = _res_chain(x_ref[...], res, (9, 3, 1), nb, 64)
    o_ref[...] = _up(v, wu, bu, nb, 64)


def _dec3_body(nb, x_ref, *rest):
    res = rest[:12]
    wu, bu, w1, b1, w2, b2 = rest[12:18]
    o_ref = rest[18]
    v = _res_chain(x_ref[...], res, (9, 3, 1), nb, 128)
    v = _up(v, wu, bu, nb, 128)
    v = jnp.maximum(_conv3(v, w1, b1, 1, nb, 256), 0.0)
    o_ref[...] = _conv3(v, w2, b2, 1, nb, 256)


def _const_spec(w):
    nd = w.ndim
    return pl.BlockSpec(w.shape, (lambda nd: (lambda i: (0,) * nd))(nd))


def _seg(body, x, weights, T_out, C_out, nb):
    B = x.shape[0]
    in_specs = [pl.BlockSpec((nb,) + x.shape[1:], lambda i: (i, 0, 0))]
    in_specs += [_const_spec(w) for w in weights]
    return pl.pallas_call(
        functools.partial(body, nb),
        grid=(B // nb,),
        in_specs=in_specs,
        out_specs=pl.BlockSpec((nb, T_out, C_out), lambda i: (i, 0, 0)),
        out_shape=jax.ShapeDtypeStruct((B, T_out, C_out), F32),
    )(x, *weights)


def _lfq_call(x2d, weights):
    n = x2d.shape[0]
    in_specs = [pl.BlockSpec(x2d.shape, lambda: (0, 0))]
    in_specs += [pl.BlockSpec(w.shape, (lambda nd: (lambda: (0,) * nd))(w.ndim))
                 for w in weights]
    return pl.pallas_call(
        _lfq_body,
        in_specs=in_specs,
        out_specs=[pl.BlockSpec(x2d.shape, lambda: (0, 0)),
                   pl.BlockSpec((1, NUM_Q), lambda: (0, 0))],
        out_shape=[jax.ShapeDtypeStruct((n, 512), F32),
                   jax.ShapeDtypeStruct((1, NUM_Q), F32)],
    )(x2d, *weights)


def _wt(p):
    return p['w'].transpose(2, 1, 0)


def _bt(p):
    return p['b'].reshape(1, -1)


def _res_w(blocks):
    out = []
    for rb in blocks:
        out += [_wt(rb['c1']), _bt(rb['c1']), _wt(rb['c2']), _bt(rb['c2'])]
    return out


def _up_w(p):
    wt = _wt(p)
    return jnp.stack([wt[0], wt[1] + wt[2], wt[0] + wt[1], wt[2]])


def kernel(x, params):
    enc, dec, lfq = params['encoder'], params['decoder'], params['lfq']
    B = x.shape[0]

    x_in = jnp.transpose(x, (0, 2, 1)).astype(F32)
    x_enc = _encoder_x(x_in, enc)
    x_enc = jnp.transpose(x_enc, (0, 2, 1))

    lfq_w = []
    for p in lfq:
        lfq_w += [p['win'].T, p['bin'].reshape(1, -1),
                  p['wout'].T, p['bout'].reshape(1, -1)]
    x_q2d, losses = _lfq_call(x_enc.reshape(B * 32, 512), lfq_w)
    x_q = x_q2d.reshape(B, 32, 512)
    commit_loss = losses.reshape(NUM_Q)
    perplexity = jnp.sum(commit_loss)

    v = _seg(_dec1_body, x_q,
             [_wt(dec['conv_in']), _bt(dec['conv_in'])]
             + _res_w(dec['ups'][0]['res'])
             + [_up_w(dec['ups'][0]['up']), _bt(dec['ups'][0]['up'])], 64, 512, 8)
    v = _seg(_dec2_body, v,
             _res_w(dec['ups'][1]['res'])
             + [_up_w(dec['ups'][1]['up']), _bt(dec['ups'][1]['up'])], 128, 512, 8)
    v = _seg(_dec3_body, v,
             _res_w(dec['ups'][2]['res'])
             + [_up_w(dec['ups'][2]['up']), _bt(dec['ups'][2]['up']),
                _wt(dec['conv1']), _bt(dec['conv1']),
                _wt(dec['conv2']), _bt(dec['conv2'])], 256, 263, 8)

    x_out = jnp.transpose(v, (0, 2, 1))
    return x_out, commit_loss, perplexity


# speedup vs baseline: 1.4679x; 1.0097x over previous
import functools

import jax
import jax.numpy as jnp
from jax.experimental import pallas as pl

F32 = jnp.float32
NUM_Q = 4
STRIDE_T = 2
DGR = 3
DEPTH = 3


def _conv1d_x(x, p, stride=1, padding=0, dilation=1):
    out = jax.lax.conv_general_dilated(x, p['w'], (stride,), [(padding, padding)],
                                       rhs_dilation=(dilation,),
                                       dimension_numbers=('NCH', 'OIH', 'NCH'))
    return out + p['b'][None, :, None]


def _resblock_x(x, p, dilation):
    h = jax.nn.relu(x)
    h = _conv1d_x(h, p['c1'], 1, dilation, dilation)
    h = jax.nn.relu(h)
    h = _conv1d_x(h, p['c2'], 1, 0, 1)
    return x + h


def _encoder_x(x, ep):
    x = jax.nn.relu(_conv1d_x(x, ep['conv_in'], 1, 1))
    for blk in ep['downs']:
        x = _conv1d_x(x, blk['down'], STRIDE_T, STRIDE_T // 2)
        for p, dil in zip(blk['res'], [DGR ** d for d in range(DEPTH)]):
            x = _resblock_x(x, p, dil)
    x = _conv1d_x(x, ep['conv_out'], 1, 1)
    return x


def _dot(a, b):
    return jax.lax.dot_general(a, b, (((1,), (0,)), ((), ())),
                               preferred_element_type=F32)


def _conv3(v, wref, bref, dil, nb, T):
    C = wref.shape[-1]
    xp = jnp.pad(v, ((0, 0), (dil, dil), (0, 0)))
    acc = jnp.broadcast_to(bref[...], (nb * T, C)).astype(F32)
    for k in range(3):
        acc = acc + _dot(xp[:, k * dil:k * dil + T, :].reshape(nb * T, -1), wref[k])
    return acc.reshape(nb, T, C)


def _conv1(v, wref, bref, nb, T):
    C = wref.shape[-1]
    return (_dot(v.reshape(nb * T, -1), wref[0]) + bref[...]).reshape(nb, T, C)


def _up(v, wref, bref, nb, T):
    C = wref.shape[-1]
    xp = jnp.pad(v, ((0, 0), (1, 1), (0, 0)))
    b = jnp.broadcast_to(bref[...], (nb * T, C)).astype(F32)
    x0 = xp[:, 0:T, :].reshape(nb * T, -1)
    x1 = xp[:, 1:T + 1, :].reshape(nb * T, -1)
    x2 = xp[:, 2:T + 2, :].reshape(nb * T, -1)
    ye = (b + _dot(x0, wref[0]) + _dot(x1, wref[1])).reshape(nb, T, 1, C)
    yo = (b + _dot(x1, wref[2]) + _dot(x2, wref[3])).reshape(nb, T, 1, C)
    return jnp.concatenate([ye, yo], axis=2).reshape(nb, 2 * T, C)


def _resblock(v, w1, b1, w2, b2, dil, nb, T):
    h = jnp.maximum(v, 0.0)
    h = _conv3(h, w1, b1, dil, nb, T)
    h = jnp.maximum(h, 0.0)
    h = _conv1(h, w2, b2, nb, T)
    return v + h


def _res_chain(v, refs, dils, nb, T):
    for i, d in enumerate(dils):
        v = _resblock(v, refs[4 * i], refs[4 * i + 1], refs[4 * i + 2],
                      refs[4 * i + 3], d, nb, T)
    return v


def _lfq_body(x_ref, *rest):
    lw = rest[:4 * NUM_Q]
    q_ref, loss_ref = rest[4 * NUM_Q], rest[4 * NUM_Q + 1]
    r = x_ref[...]
    acc = jnp.zeros_like(r)
    for l in range(NUM_Q):
        winT, bin_, woutT, bout = lw[4 * l:4 * l + 4]
        z = _dot(r, winT[...]) + bin_[...]
        q = jnp.where(z > 0, 1.0, -1.0).astype(F32)
        out = _dot(q, woutT[...]) + bout[...]
        r = r - out
        acc = acc + out
        loss_ref[0:1, l:l + 1] = jnp.mean((z - q) ** 2).reshape(1, 1)
    q_ref[...] = acc


def _dec1_body(nb, x_ref, wi, bi, *rest):
    res = rest[:12]
    wu, bu = rest[12], rest[13]
    o_ref = rest[14]
    v = jnp.maximum(_conv3(x_ref[...], wi, bi, 1, nb, 32), 0.0)
    v = _res_chain(v, res, (9, 3, 1), nb, 32)
    o_ref[...] = _up(v, wu, bu, nb, 32)


def _dec2_body(nb, x_ref, *rest):
    res = rest[:12]
    wu, bu = rest[12], rest[13]
    o_ref = rest[14]
    v = _res_chain(x_ref[...], res, (9, 3, 1), nb, 64)
    o_ref[...] = _up(v, wu, bu, nb, 64)


def _dec3_body(nb, x_ref, *rest):
    res = rest[:12]
    wu, bu, w1, b1, w2, b2 = rest[12:18]
    o_ref = rest[18]
    v = _res_chain(x_ref[...], res, (9, 3, 1), nb, 128)
    v = _up(v, wu, bu, nb, 128)
    v = jnp.maximum(_conv3(v, w1, b1, 1, nb, 256), 0.0)
    o_ref[...] = _conv3(v, w2, b2, 1, nb, 256)


def _const_spec(w):
    nd = w.ndim
    return pl.BlockSpec(w.shape, (lambda nd: (lambda i: (0,) * nd))(nd))


def _seg(body, x, weights, T_out, C_out, nb):
    B = x.shape[0]
    in_specs = [pl.BlockSpec((nb,) + x.shape[1:], lambda i: (i, 0, 0))]
    in_specs += [_const_spec(w) for w in weights]
    return pl.pallas_call(
        functools.partial(body, nb),
        grid=(B // nb,),
        in_specs=in_specs,
        out_specs=pl.BlockSpec((nb, T_out, C_out), lambda i: (i, 0, 0)),
        out_shape=jax.ShapeDtypeStruct((B, T_out, C_out), F32),
    )(x, *weights)


def _lfq_call(x2d, weights):
    n = x2d.shape[0]
    in_specs = [pl.BlockSpec(x2d.shape, lambda: (0, 0))]
    in_specs += [pl.BlockSpec(w.shape, (lambda nd: (lambda: (0,) * nd))(w.ndim))
                 for w in weights]
    return pl.pallas_call(
        _lfq_body,
        in_specs=in_specs,
        out_specs=[pl.BlockSpec(x2d.shape, lambda: (0, 0)),
                   pl.BlockSpec((1, NUM_Q), lambda: (0, 0))],
        out_shape=[jax.ShapeDtypeStruct((n, 512), F32),
                   jax.ShapeDtypeStruct((1, NUM_Q), F32)],
    )(x2d, *weights)


def _wt(p):
    return p['w'].transpose(2, 1, 0)


def _bt(p):
    return p['b'].reshape(1, -1)


def _res_w(blocks):
    out = []
    for rb in blocks:
        out += [_wt(rb['c1']), _bt(rb['c1']), _wt(rb['c2']), _bt(rb['c2'])]
    return out


def _up_w(p):
    wt = _wt(p)
    return jnp.stack([wt[0], wt[1] + wt[2], wt[0] + wt[1], wt[2]])


def kernel(x, params):
    enc, dec, lfq = params['encoder'], params['decoder'], params['lfq']
    B = x.shape[0]

    x_in = jnp.transpose(x, (0, 2, 1)).astype(F32)
    x_enc = _encoder_x(x_in, enc)
    x_enc = jnp.transpose(x_enc, (0, 2, 1))

    lfq_w = []
    for p in lfq:
        lfq_w += [p['win'].T, p['bin'].reshape(1, -1),
                  p['wout'].T, p['bout'].reshape(1, -1)]
    x_q2d, losses = _lfq_call(x_enc.reshape(B * 32, 512), lfq_w)
    x_q = x_q2d.reshape(B, 32, 512)
    commit_loss = losses.reshape(NUM_Q)
    perplexity = jnp.sum(commit_loss)

    v = _seg(_dec1_body, x_q,
             [_wt(dec['conv_in']), _bt(dec['conv_in'])]
             + _res_w(dec['ups'][0]['res'])
             + [_up_w(dec['ups'][0]['up']), _bt(dec['ups'][0]['up'])], 64, 512, 16)
    v = _seg(_dec2_body, v,
             _res_w(dec['ups'][1]['res'])
             + [_up_w(dec['ups'][1]['up']), _bt(dec['ups'][1]['up'])], 128, 512, 16)
    v = _seg(_dec3_body, v,
             _res_w(dec['ups'][2]['res'])
             + [_up_w(dec['ups'][2]['up']), _bt(dec['ups'][2]['up']),
                _wt(dec['conv1']), _bt(dec['conv1']),
                _wt(dec['conv2']), _bt(dec['conv2'])], 256, 263, 8)

    x_out = jnp.transpose(v, (0, 2, 1))
    return x_out, commit_loss, perplexity
